# baseline (speedup 1.0000x reference)
import jax
import jax.numpy as jnp
from jax.experimental import pallas as pl
from jax.experimental.pallas import tpu as pltpu

N_NODES = 50000
K_NB = 31
D = 50
D_ENT = 128
KP = 56
BLK = 1024

_BF = jnp.bfloat16
_F32 = jnp.float32


_KC = 16
_DC = 10


def _body(qT_ref, ent_ref, paraT_ref, scoreT_ref, B2T_ref, Went_ref,
          bent_ref, WgrnA_ref, Waux_ref, Wrank_ref, brank_ref, out_ref):
    tT = jnp.dot(B2T_ref[...], qT_ref[...],
                 preferred_element_type=_F32)
    eb = ent_ref[...].astype(_BF)
    entT = jax.lax.dot_general(Went_ref[...], eb, (((1,), (1,)), ((), ())),
                               preferred_element_type=_F32)
    entT = entT + bent_ref[...]
    nodeT = entT[0:1, :] * tT[0:D, :]
    for j in range(1, D):
        nodeT = nodeT + entT[j:j + 1, :] * tT[j * KP:j * KP + D, :]

    score = scoreT_ref[...]
    s31 = score[K_NB:K_NB + 1, :]
    wpara = jnp.sum(paraT_ref[...] * score[None, :K_NB, :], axis=1)
    w = wpara + s31 * nodeT
    aux = jnp.concatenate([s31, jnp.ones((1, w.shape[1]), _F32)], axis=0)
    on = (jnp.dot(WgrnA_ref[...], w.astype(_BF), preferred_element_type=_F32)
          + jnp.dot(Waux_ref[...], aux.astype(_BF),
                    preferred_element_type=_F32))
    on = jnp.where(on > 0, on, jnp.exp(jnp.minimum(on, 0.0)) - 1.0)
    out_ref[...] = jnp.dot(Wrank_ref[...], on.astype(_BF),
                           preferred_element_type=_F32) + brank_ref[...]


@jax.jit
def kernel(query_emb, entity_emb, neighbors_para, neighbors_score, W_ent,
           b_ent, B_bil, b_bil, W_grn, grn_bias, W_rank, b_rank):
    qT = query_emb.T.astype(_BF)
    scoreT = neighbors_score.T
    paraT = jnp.transpose(neighbors_para, (2, 1, 0))
    B2T = jnp.transpose(B_bil, (2, 0, 1))
    B2T = jnp.pad(B2T, ((0, 0), (0, KP - D), (0, 0))).reshape(D * KP, D)
    Waux = jnp.concatenate([(W_grn @ b_bil)[:, None], grn_bias[:, None]],
                           axis=1)

    grid = (pl.cdiv(N_NODES, BLK),)
    c0 = lambda i: (0, i)
    w0 = lambda i: (0, 0)
    outT = pl.pallas_call(
        _body,
        grid=grid,
        in_specs=[
            pl.BlockSpec((D, BLK), c0),
            pl.BlockSpec((BLK, D_ENT), lambda i: (i, 0)),
            pl.BlockSpec((D, K_NB, BLK), lambda i: (0, 0, i)),
            pl.BlockSpec((K_NB + 1, BLK), c0),
            pl.BlockSpec((D * KP, D), w0),
            pl.BlockSpec((D, D_ENT), w0),
            pl.BlockSpec((D, 1), w0),
            pl.BlockSpec((D, D), w0),
            pl.BlockSpec((D, 2), w0),
            pl.BlockSpec((1, D), w0),
            pl.BlockSpec((1, 1), w0),
        ],
        out_specs=pl.BlockSpec((1, BLK), c0),
        out_shape=jax.ShapeDtypeStruct((1, N_NODES), _F32),
        compiler_params=pltpu.CompilerParams(
            dimension_semantics=("arbitrary",)),
    )(qT, entity_emb, paraT, scoreT,
      B2T.astype(_BF), W_ent.astype(_BF), b_ent[:, None],
      W_grn.astype(_BF), Waux.astype(_BF), W_rank.astype(_BF),
      b_rank[:, None])
    return outT.T


# speedup vs baseline: 1.0267x; 1.0267x over previous
import jax
import jax.numpy as jnp
from jax.experimental import pallas as pl
from jax.experimental.pallas import tpu as pltpu

N_NODES = 50000
K_NB = 31
D = 50
D_ENT = 128
KP = 56
BLK = 1024

_BF = jnp.bfloat16
_F32 = jnp.float32


_KC = 16
_DC = 10


def _body(qT_ref, ent_ref, paraT_ref, scoreT_ref, B2T_ref, Went_ref,
          bent_ref, WgrnA_ref, Waux_ref, Wrank_ref, brank_ref, out_ref):
    qb = qT_ref[...].astype(_BF)
    tT = jnp.dot(B2T_ref[...], qb, preferred_element_type=_F32)
    eb = ent_ref[...].astype(_BF)
    entT = jax.lax.dot_general(Went_ref[...], eb, (((1,), (1,)), ((), ())),
                               preferred_element_type=_F32)
    entT = entT + bent_ref[...]
    nodeT = entT[0:1, :] * tT[0:D, :]
    for j in range(1, D):
        nodeT = nodeT + entT[j:j + 1, :] * tT[j * KP:j * KP + D, :]

    score = scoreT_ref[...]
    s31 = score[K_NB:K_NB + 1, :]
    wpara = jnp.sum(paraT_ref[...] * score[None, :K_NB, :], axis=1)
    w = wpara + s31 * nodeT
    aux = jnp.concatenate([s31, jnp.ones((1, w.shape[1]), _F32)], axis=0)
    on = (jnp.dot(WgrnA_ref[...], w.astype(_BF), preferred_element_type=_F32)
          + jnp.dot(Waux_ref[...], aux.astype(_BF),
                    preferred_element_type=_F32))
    on = jnp.where(on > 0, on, jnp.exp(jnp.minimum(on, 0.0)) - 1.0)
    out_ref[...] = jnp.dot(Wrank_ref[...], on.astype(_BF),
                           preferred_element_type=_F32) + brank_ref[...]


@jax.jit
def kernel(query_emb, entity_emb, neighbors_para, neighbors_score, W_ent,
           b_ent, B_bil, b_bil, W_grn, grn_bias, W_rank, b_rank):
    qT = query_emb.T
    scoreT = neighbors_score.T
    paraT = jnp.transpose(neighbors_para, (2, 1, 0))
    B2T = jnp.transpose(B_bil, (2, 0, 1))
    B2T = jnp.pad(B2T, ((0, 0), (0, KP - D), (0, 0))).reshape(D * KP, D)
    Waux = jnp.concatenate([(W_grn @ b_bil)[:, None], grn_bias[:, None]],
                           axis=1)

    grid = (pl.cdiv(N_NODES, BLK),)
    c0 = lambda i: (0, i)
    w0 = lambda i: (0, 0)
    outT = pl.pallas_call(
        _body,
        grid=grid,
        in_specs=[
            pl.BlockSpec((D, BLK), c0),
            pl.BlockSpec((BLK, D_ENT), lambda i: (i, 0)),
            pl.BlockSpec((D, K_NB, BLK), lambda i: (0, 0, i)),
            pl.BlockSpec((K_NB + 1, BLK), c0),
            pl.BlockSpec((D * KP, D), w0),
            pl.BlockSpec((D, D_ENT), w0),
            pl.BlockSpec((D, 1), w0),
            pl.BlockSpec((D, D), w0),
            pl.BlockSpec((D, 2), w0),
            pl.BlockSpec((1, D), w0),
            pl.BlockSpec((1, 1), w0),
        ],
        out_specs=pl.BlockSpec((1, BLK), c0),
        out_shape=jax.ShapeDtypeStruct((1, N_NODES), _F32),
        compiler_params=pltpu.CompilerParams(
            dimension_semantics=("parallel",)),
    )(qT, entity_emb, paraT, scoreT,
      B2T.astype(_BF), W_ent.astype(_BF), b_ent[:, None],
      W_grn.astype(_BF), Waux.astype(_BF), W_rank.astype(_BF),
      b_rank[:, None])
    return outT.T
